# baseline (speedup 1.0000x reference)
import functools

import jax
import jax.numpy as jnp
from jax import lax
from jax.experimental import pallas as pl
from jax.experimental.pallas import tpu as pltpu
from jax.experimental.pallas import tpu_sc as plsc

NUM_ROWS = 1000000
N_FACTORS = 32
BATCH = 16384

_info = plsc.get_sparse_core_info()
NC, NS, L = _info.num_cores, _info.num_subcores, _info.num_lanes
NW = NC * NS
BPW = BATCH // NW
CHUNK = 128
N_CHUNKS = BPW // CHUNK

TBLK = 4096


def _transpose_body(t_ref, out_ref):
    row = lax.broadcasted_iota(jnp.int32, (N_FACTORS, N_FACTORS), 0)
    col = lax.broadcasted_iota(jnp.int32, (N_FACTORS, N_FACTORS), 1)
    ident = jnp.where(row == col, 1.0, 0.0).astype(jnp.float32)
    out_ref[...] = lax.dot_general(
        t_ref[...], ident, (((0,), (0,)), ((), ())),
        preferred_element_type=jnp.float32)


def _relayout(table_t):
    grid = (NUM_ROWS + TBLK - 1) // TBLK
    return pl.pallas_call(
        _transpose_body,
        grid=(grid,),
        in_specs=[pl.BlockSpec((N_FACTORS, TBLK), lambda i: (0, i))],
        out_specs=pl.BlockSpec((TBLK, N_FACTORS), lambda i: (i, 0)),
        out_shape=jax.ShapeDtypeStruct((NUM_ROWS, N_FACTORS), jnp.float32),
    )(table_t)


def _mf_body(uf_hbm, if_hbm, users_hbm, items_hbm, out_hbm,
             uidx_v, iidx_v, urows_v, irows_v, out_v, sem):
    wid = lax.axis_index("s") * NC + lax.axis_index("c")
    base = wid * BPW

    pltpu.sync_copy(users_hbm.at[pl.ds(base, BPW)], uidx_v)
    pltpu.sync_copy(items_hbm.at[pl.ds(base, BPW)], iidx_v)

    copies = []
    for k in range(N_CHUNKS):
        sl = pl.ds(k * CHUNK, CHUNK)
        copies.append(pltpu.async_copy(uf_hbm.at[uidx_v.at[sl]],
                                       urows_v.at[sl], sem))
        copies.append(pltpu.async_copy(if_hbm.at[iidx_v.at[sl]],
                                       irows_v.at[sl], sem))
    for c in copies:
        c.wait()

    lane = lax.iota(jnp.int32, L)

    def group_body(g, _):
        rows = g * L + lane
        acc = jnp.zeros((L,), jnp.float32)
        for d in range(N_FACTORS):
            col = jnp.full((L,), d, jnp.int32)
            uu = plsc.load_gather(urows_v, [rows, col])
            vv = plsc.load_gather(irows_v, [rows, col])
            acc = acc + uu * vv
        out_v[pl.ds(g * L, L)] = acc
        return 0

    lax.fori_loop(0, BPW // L, group_body, 0)

    pltpu.sync_copy(out_v, out_hbm.at[pl.ds(base, BPW)])


@jax.jit
def kernel(user_factors, item_factors, users, items):
    users = users.astype(jnp.int32)
    items = items.astype(jnp.int32)
    uf_lin = _relayout(user_factors.T)
    if_lin = _relayout(item_factors.T)
    mesh = plsc.VectorSubcoreMesh(core_axis_name="c", subcore_axis_name="s")
    run = pl.kernel(
        _mf_body,
        out_type=jax.ShapeDtypeStruct((BATCH,), jnp.float32),
        mesh=mesh,
        scratch_types=[
            pltpu.VMEM((BPW,), jnp.int32),
            pltpu.VMEM((BPW,), jnp.int32),
            pltpu.VMEM((BPW, N_FACTORS), jnp.float32),
            pltpu.VMEM((BPW, N_FACTORS), jnp.float32),
            pltpu.VMEM((BPW,), jnp.float32),
            pltpu.SemaphoreType.DMA,
        ],
        compiler_params=pltpu.CompilerParams(
            needs_layout_passes=False, use_tc_tiling_on_sc=False),
    )
    return run(uf_lin, if_lin, users, items)


# speedup vs baseline: 1.2220x; 1.2220x over previous
import functools

import jax
import jax.numpy as jnp
from jax import lax
from jax.experimental import pallas as pl
from jax.experimental.pallas import tpu as pltpu
from jax.experimental.pallas import tpu_sc as plsc

NUM_ROWS = 1000000
N_FACTORS = 32
BATCH = 16384

_info = plsc.get_sparse_core_info()
NC, NS, L = _info.num_cores, _info.num_subcores, _info.num_lanes
NW = NC * NS
BPW = BATCH // NW
CHUNK = 128
N_CHUNKS = BPW // CHUNK

TBLK = 16384


def _transpose_body(u_ref, v_ref, uo_ref, vo_ref):
    row = lax.broadcasted_iota(jnp.int32, (N_FACTORS, N_FACTORS), 0)
    col = lax.broadcasted_iota(jnp.int32, (N_FACTORS, N_FACTORS), 1)
    ident = jnp.where(row == col, 1.0, 0.0).astype(jnp.float32)
    dn = (((0,), (0,)), ((), ()))
    uo_ref[...] = lax.dot_general(u_ref[...], ident, dn,
                                  preferred_element_type=jnp.float32)
    vo_ref[...] = lax.dot_general(v_ref[...], ident, dn,
                                  preferred_element_type=jnp.float32)


def _relayout(uf_t, if_t):
    grid = (NUM_ROWS + TBLK - 1) // TBLK
    ispec = pl.BlockSpec((N_FACTORS, TBLK), lambda i: (0, i))
    ospec = pl.BlockSpec((TBLK, N_FACTORS), lambda i: (i, 0))
    oshape = jax.ShapeDtypeStruct((NUM_ROWS, N_FACTORS), jnp.float32)
    return pl.pallas_call(
        _transpose_body,
        grid=(grid,),
        in_specs=[ispec, ispec],
        out_specs=[ospec, ospec],
        out_shape=[oshape, oshape],
    )(uf_t, if_t)


def _mf_body(uf_hbm, if_hbm, users_hbm, items_hbm, out_hbm,
             uidx_v, iidx_v, urows_v, irows_v, out_v, sem):
    wid = lax.axis_index("s") * NC + lax.axis_index("c")
    base = wid * BPW

    pltpu.sync_copy(users_hbm.at[pl.ds(base, BPW)], uidx_v)
    pltpu.sync_copy(items_hbm.at[pl.ds(base, BPW)], iidx_v)

    copies = []
    for k in range(N_CHUNKS):
        sl = pl.ds(k * CHUNK, CHUNK)
        copies.append(pltpu.async_copy(uf_hbm.at[uidx_v.at[sl]],
                                       urows_v.at[sl], sem))
        copies.append(pltpu.async_copy(if_hbm.at[iidx_v.at[sl]],
                                       irows_v.at[sl], sem))
    for c in copies:
        c.wait()

    lane = lax.iota(jnp.int32, L)

    def group_body(g, _):
        rows = g * L + lane
        acc = jnp.zeros((L,), jnp.float32)
        for d in range(N_FACTORS):
            col = jnp.full((L,), d, jnp.int32)
            uu = plsc.load_gather(urows_v, [rows, col])
            vv = plsc.load_gather(irows_v, [rows, col])
            acc = acc + uu * vv
        out_v[pl.ds(g * L, L)] = acc
        return 0

    lax.fori_loop(0, BPW // L, group_body, 0)

    pltpu.sync_copy(out_v, out_hbm.at[pl.ds(base, BPW)])


@jax.jit
def kernel(user_factors, item_factors, users, items):
    users = users.astype(jnp.int32)
    items = items.astype(jnp.int32)
    uf_lin, if_lin = _relayout(user_factors.T, item_factors.T)
    mesh = plsc.VectorSubcoreMesh(core_axis_name="c", subcore_axis_name="s")
    run = pl.kernel(
        _mf_body,
        out_type=jax.ShapeDtypeStruct((BATCH,), jnp.float32),
        mesh=mesh,
        scratch_types=[
            pltpu.VMEM((BPW,), jnp.int32),
            pltpu.VMEM((BPW,), jnp.int32),
            pltpu.VMEM((BPW, N_FACTORS), jnp.float32),
            pltpu.VMEM((BPW, N_FACTORS), jnp.float32),
            pltpu.VMEM((BPW,), jnp.float32),
            pltpu.SemaphoreType.DMA,
        ],
        compiler_params=pltpu.CompilerParams(
            needs_layout_passes=False, use_tc_tiling_on_sc=False),
    )
    return run(uf_lin, if_lin, users, items)


# speedup vs baseline: 2.5689x; 2.1021x over previous
import jax
import jax.numpy as jnp
from jax import lax
from jax.experimental import pallas as pl
from jax.experimental.pallas import tpu as pltpu
from jax.experimental.pallas import tpu_sc as plsc

NUM_ROWS = 1000000
N_FACTORS = 32
BATCH = 16384
PACK = 128 // N_FACTORS
NBLK = (NUM_ROWS + 16384 - 1) // 16384
NLINES = NBLK * (16384 // PACK)

_info = plsc.get_sparse_core_info()
NC, NS, L = _info.num_cores, _info.num_subcores, _info.num_lanes
NW = NC * NS
BPW = BATCH // NW
CHUNK = 128
HALF = 256

TBLK = 16384


def _transpose_body(u_ref, v_ref, uo_ref, vo_ref):
    row = lax.broadcasted_iota(jnp.int32, (N_FACTORS, N_FACTORS), 0)
    col = lax.broadcasted_iota(jnp.int32, (N_FACTORS, N_FACTORS), 1)
    ident = jnp.where(row == col, 1.0, 0.0).astype(jnp.float32)
    dn = (((0,), (0,)), ((), ()))
    sub = TBLK // PACK
    u = u_ref[...]
    v = v_ref[...]
    for a in range(PACK):
        us = lax.slice(u, (0, a * sub), (N_FACTORS, (a + 1) * sub))
        vs = lax.slice(v, (0, a * sub), (N_FACTORS, (a + 1) * sub))
        uo_ref[:, a * N_FACTORS:(a + 1) * N_FACTORS] = lax.dot_general(
            us, ident, dn, preferred_element_type=jnp.float32)
        vo_ref[:, a * N_FACTORS:(a + 1) * N_FACTORS] = lax.dot_general(
            vs, ident, dn, preferred_element_type=jnp.float32)


def _relayout(uf_t, if_t):
    grid = (NUM_ROWS + TBLK - 1) // TBLK
    ispec = pl.BlockSpec((N_FACTORS, TBLK), lambda i: (0, i))
    ospec = pl.BlockSpec((TBLK // PACK, PACK * N_FACTORS), lambda i: (i, 0))
    oshape = jax.ShapeDtypeStruct((NLINES, PACK * N_FACTORS), jnp.float32)
    return pl.pallas_call(
        _transpose_body,
        grid=(grid,),
        in_specs=[ispec, ispec],
        out_specs=[ospec, ospec],
        out_shape=[oshape, oshape],
    )(uf_t, if_t)


def _mf_body(uf_hbm, if_hbm, users_hbm, items_hbm, out_hbm,
             uidx_v, iidx_v, ulns_v, ilns_v, urow_v, irow_v, out_v, sem):
    wid = lax.axis_index("s") * NC + lax.axis_index("c")
    base = wid * BPW

    pltpu.sync_copy(users_hbm.at[pl.ds(base, BPW)], uidx_v)
    pltpu.sync_copy(items_hbm.at[pl.ds(base, BPW)], iidx_v)

    def line_body(j, _):
        sl = pl.ds(j * L, L)
        u = uidx_v[sl]
        v = iidx_v[sl]
        ulns_v[sl] = (u >> 14) * 4096 + (u & 4095)
        ilns_v[sl] = (v >> 14) * 4096 + (v & 4095)
        return 0
    lax.fori_loop(0, BPW // L, line_body, 0)

    lane = lax.iota(jnp.int32, L)

    def half_body(h, _):
        hbase = h * HALF
        copies = []
        for k in range(HALF // CHUNK):
            src = pl.ds(hbase + k * CHUNK, CHUNK)
            dst = pl.ds(k * CHUNK, CHUNK)
            copies.append(pltpu.async_copy(uf_hbm.at[ulns_v.at[src]],
                                           urow_v.at[dst], sem))
            copies.append(pltpu.async_copy(if_hbm.at[ilns_v.at[src]],
                                           irow_v.at[dst], sem))
        for c in copies:
            c.wait()

        def group_body(g, _):
            rows = g * L + lane
            ucol = ((uidx_v[pl.ds(hbase + g * L, L)] >> 12) & 3) * N_FACTORS
            icol = ((iidx_v[pl.ds(hbase + g * L, L)] >> 12) & 3) * N_FACTORS
            acc = jnp.zeros((L,), jnp.float32)
            for d in range(N_FACTORS):
                uu = plsc.load_gather(urow_v, [rows, ucol + d])
                vv = plsc.load_gather(irow_v, [rows, icol + d])
                acc = acc + uu * vv
            out_v[pl.ds(hbase + g * L, L)] = acc
            return 0

        lax.fori_loop(0, HALF // L, group_body, 0)
        return 0

    lax.fori_loop(0, BPW // HALF, half_body, 0)

    pltpu.sync_copy(out_v, out_hbm.at[pl.ds(base, BPW)])


@jax.jit
def kernel(user_factors, item_factors, users, items):
    users = users.astype(jnp.int32)
    items = items.astype(jnp.int32)
    uf_lin, if_lin = _relayout(user_factors.T, item_factors.T)
    mesh = plsc.VectorSubcoreMesh(core_axis_name="c", subcore_axis_name="s")
    run = pl.kernel(
        _mf_body,
        out_type=jax.ShapeDtypeStruct((BATCH,), jnp.float32),
        mesh=mesh,
        scratch_types=[
            pltpu.VMEM((BPW,), jnp.int32),
            pltpu.VMEM((BPW,), jnp.int32),
            pltpu.VMEM((BPW,), jnp.int32),
            pltpu.VMEM((BPW,), jnp.int32),
            pltpu.VMEM((HALF, PACK * N_FACTORS), jnp.float32),
            pltpu.VMEM((HALF, PACK * N_FACTORS), jnp.float32),
            pltpu.VMEM((BPW,), jnp.float32),
            pltpu.SemaphoreType.DMA,
        ],
        compiler_params=pltpu.CompilerParams(
            needs_layout_passes=False, use_tc_tiling_on_sc=False),
    )
    return run(uf_lin, if_lin, users, items)


# speedup vs baseline: 2.5766x; 1.0030x over previous
import jax
import jax.numpy as jnp
from jax import lax
from jax.experimental import pallas as pl
from jax.experimental.pallas import tpu as pltpu
from jax.experimental.pallas import tpu_sc as plsc

NUM_ROWS = 1000000
N_FACTORS = 32
BATCH = 16384
PACK = 128 // N_FACTORS
NBLK = (NUM_ROWS + 32768 - 1) // 32768
NLINES = NBLK * (32768 // PACK)

_info = plsc.get_sparse_core_info()
NC, NS, L = _info.num_cores, _info.num_subcores, _info.num_lanes
NW = NC * NS
BPW = BATCH // NW
CHUNK = 128
HALF = 256

TBLK = 32768


def _transpose_body(u_ref, v_ref, uo_ref, vo_ref):
    row = lax.broadcasted_iota(jnp.int32, (N_FACTORS, N_FACTORS), 0)
    col = lax.broadcasted_iota(jnp.int32, (N_FACTORS, N_FACTORS), 1)
    ident = jnp.where(row == col, 1.0, 0.0).astype(jnp.float32)
    dn = (((0,), (0,)), ((), ()))
    sub = TBLK // PACK
    u = u_ref[...]
    v = v_ref[...]
    for a in range(PACK):
        us = lax.slice(u, (0, a * sub), (N_FACTORS, (a + 1) * sub))
        vs = lax.slice(v, (0, a * sub), (N_FACTORS, (a + 1) * sub))
        uo_ref[:, a * N_FACTORS:(a + 1) * N_FACTORS] = lax.dot_general(
            us, ident, dn, preferred_element_type=jnp.float32)
        vo_ref[:, a * N_FACTORS:(a + 1) * N_FACTORS] = lax.dot_general(
            vs, ident, dn, preferred_element_type=jnp.float32)


def _relayout(uf_t, if_t):
    grid = (NUM_ROWS + TBLK - 1) // TBLK
    ispec = pl.BlockSpec((N_FACTORS, TBLK), lambda i: (0, i))
    ospec = pl.BlockSpec((TBLK // PACK, PACK * N_FACTORS), lambda i: (i, 0))
    oshape = jax.ShapeDtypeStruct((NLINES, PACK * N_FACTORS), jnp.float32)
    return pl.pallas_call(
        _transpose_body,
        grid=(grid,),
        in_specs=[ispec, ispec],
        out_specs=[ospec, ospec],
        out_shape=[oshape, oshape],
    )(uf_t, if_t)


def _mf_body(uf_hbm, if_hbm, users_hbm, items_hbm, out_hbm,
             uidx_v, iidx_v, ulns_v, ilns_v, urow_v, irow_v, out_v, sem):
    wid = lax.axis_index("s") * NC + lax.axis_index("c")
    base = wid * BPW

    pltpu.sync_copy(users_hbm.at[pl.ds(base, BPW)], uidx_v)
    pltpu.sync_copy(items_hbm.at[pl.ds(base, BPW)], iidx_v)

    def line_body(j, _):
        sl = pl.ds(j * L, L)
        u = uidx_v[sl]
        v = iidx_v[sl]
        ulns_v[sl] = (u >> 15) * 8192 + (u & 8191)
        ilns_v[sl] = (v >> 15) * 8192 + (v & 8191)
        return 0
    lax.fori_loop(0, BPW // L, line_body, 0)

    lane = lax.iota(jnp.int32, L)

    def half_body(h, _):
        hbase = h * HALF
        copies = []
        for k in range(HALF // CHUNK):
            src = pl.ds(hbase + k * CHUNK, CHUNK)
            dst = pl.ds(k * CHUNK, CHUNK)
            copies.append(pltpu.async_copy(uf_hbm.at[ulns_v.at[src]],
                                           urow_v.at[dst], sem))
            copies.append(pltpu.async_copy(if_hbm.at[ilns_v.at[src]],
                                           irow_v.at[dst], sem))
        for c in copies:
            c.wait()

        def group_body(g, _):
            rows = g * L + lane
            ucol = ((uidx_v[pl.ds(hbase + g * L, L)] >> 13) & 3) * N_FACTORS
            icol = ((iidx_v[pl.ds(hbase + g * L, L)] >> 13) & 3) * N_FACTORS
            acc = jnp.zeros((L,), jnp.float32)
            for d in range(N_FACTORS):
                uu = plsc.load_gather(urow_v, [rows, ucol + d])
                vv = plsc.load_gather(irow_v, [rows, icol + d])
                acc = acc + uu * vv
            out_v[pl.ds(hbase + g * L, L)] = acc
            return 0

        lax.fori_loop(0, HALF // L, group_body, 0)
        return 0

    lax.fori_loop(0, BPW // HALF, half_body, 0)

    pltpu.sync_copy(out_v, out_hbm.at[pl.ds(base, BPW)])


@jax.jit
def kernel(user_factors, item_factors, users, items):
    users = users.astype(jnp.int32)
    items = items.astype(jnp.int32)
    uf_lin, if_lin = _relayout(user_factors.T, item_factors.T)
    mesh = plsc.VectorSubcoreMesh(core_axis_name="c", subcore_axis_name="s")
    run = pl.kernel(
        _mf_body,
        out_type=jax.ShapeDtypeStruct((BATCH,), jnp.float32),
        mesh=mesh,
        scratch_types=[
            pltpu.VMEM((BPW,), jnp.int32),
            pltpu.VMEM((BPW,), jnp.int32),
            pltpu.VMEM((BPW,), jnp.int32),
            pltpu.VMEM((BPW,), jnp.int32),
            pltpu.VMEM((HALF, PACK * N_FACTORS), jnp.float32),
            pltpu.VMEM((HALF, PACK * N_FACTORS), jnp.float32),
            pltpu.VMEM((BPW,), jnp.float32),
            pltpu.SemaphoreType.DMA,
        ],
        compiler_params=pltpu.CompilerParams(
            needs_layout_passes=False, use_tc_tiling_on_sc=False),
    )
    return run(uf_lin, if_lin, users, items)


# speedup vs baseline: 2.5780x; 1.0005x over previous
import jax
import jax.numpy as jnp
from jax import lax
from jax.experimental import pallas as pl
from jax.experimental.pallas import tpu as pltpu
from jax.experimental.pallas import tpu_sc as plsc

NUM_ROWS = 1000000
N_FACTORS = 32
BATCH = 16384
PACK = 128 // N_FACTORS
NBLK = (NUM_ROWS + 32768 - 1) // 32768
NLINES = NBLK * (32768 // PACK)

_info = plsc.get_sparse_core_info()
NC, NS, L = _info.num_cores, _info.num_subcores, _info.num_lanes
NW = NC * NS
BPW = BATCH // NW
CHUNK = 128
HALF = 256

TBLK = 32768


def _transpose_body(u_ref, v_ref, uo_ref, vo_ref):
    row = lax.broadcasted_iota(jnp.int32, (N_FACTORS, N_FACTORS), 0)
    col = lax.broadcasted_iota(jnp.int32, (N_FACTORS, N_FACTORS), 1)
    ident = jnp.where(row == col, 1.0, 0.0).astype(jnp.float32)
    dn = (((0,), (0,)), ((), ()))
    sub = TBLK // PACK
    u = u_ref[...]
    v = v_ref[...]
    for a in range(PACK):
        us = lax.slice(u, (0, a * sub), (N_FACTORS, (a + 1) * sub))
        vs = lax.slice(v, (0, a * sub), (N_FACTORS, (a + 1) * sub))
        uo_ref[:, a * N_FACTORS:(a + 1) * N_FACTORS] = lax.dot_general(
            us, ident, dn, preferred_element_type=jnp.float32)
        vo_ref[:, a * N_FACTORS:(a + 1) * N_FACTORS] = vs.T


def _relayout(uf_t, if_t):
    grid = (NUM_ROWS + TBLK - 1) // TBLK
    ispec = pl.BlockSpec((N_FACTORS, TBLK), lambda i: (0, i))
    ospec = pl.BlockSpec((TBLK // PACK, PACK * N_FACTORS), lambda i: (i, 0))
    oshape = jax.ShapeDtypeStruct((NLINES, PACK * N_FACTORS), jnp.float32)
    return pl.pallas_call(
        _transpose_body,
        grid=(grid,),
        in_specs=[ispec, ispec],
        out_specs=[ospec, ospec],
        out_shape=[oshape, oshape],
    )(uf_t, if_t)


def _mf_body(uf_hbm, if_hbm, users_hbm, items_hbm, out_hbm,
             uidx_v, iidx_v, ulns_v, ilns_v, urow_v, irow_v, out_v, sem):
    wid = lax.axis_index("s") * NC + lax.axis_index("c")
    base = wid * BPW

    pltpu.sync_copy(users_hbm.at[pl.ds(base, BPW)], uidx_v)
    pltpu.sync_copy(items_hbm.at[pl.ds(base, BPW)], iidx_v)

    def line_body(j, _):
        sl = pl.ds(j * L, L)
        u = uidx_v[sl]
        v = iidx_v[sl]
        ulns_v[sl] = (u >> 15) * 8192 + (u & 8191)
        ilns_v[sl] = (v >> 15) * 8192 + (v & 8191)
        return 0
    lax.fori_loop(0, BPW // L, line_body, 0)

    lane = lax.iota(jnp.int32, L)

    def half_body(h, _):
        hbase = h * HALF
        copies = []
        for k in range(HALF // CHUNK):
            src = pl.ds(hbase + k * CHUNK, CHUNK)
            dst = pl.ds(k * CHUNK, CHUNK)
            copies.append(pltpu.async_copy(uf_hbm.at[ulns_v.at[src]],
                                           urow_v.at[dst], sem))
            copies.append(pltpu.async_copy(if_hbm.at[ilns_v.at[src]],
                                           irow_v.at[dst], sem))
        for c in copies:
            c.wait()

        def group_body(g, _):
            rows = g * L + lane
            ucol = ((uidx_v[pl.ds(hbase + g * L, L)] >> 13) & 3) * N_FACTORS
            icol = ((iidx_v[pl.ds(hbase + g * L, L)] >> 13) & 3) * N_FACTORS
            acc = jnp.zeros((L,), jnp.float32)
            for d in range(N_FACTORS):
                uu = plsc.load_gather(urow_v, [rows, ucol + d])
                vv = plsc.load_gather(irow_v, [rows, icol + d])
                acc = acc + uu * vv
            out_v[pl.ds(hbase + g * L, L)] = acc
            return 0

        lax.fori_loop(0, HALF // L, group_body, 0)
        return 0

    lax.fori_loop(0, BPW // HALF, half_body, 0)

    pltpu.sync_copy(out_v, out_hbm.at[pl.ds(base, BPW)])


@jax.jit
def kernel(user_factors, item_factors, users, items):
    users = users.astype(jnp.int32)
    items = items.astype(jnp.int32)
    uf_lin, if_lin = _relayout(user_factors.T, item_factors.T)
    mesh = plsc.VectorSubcoreMesh(core_axis_name="c", subcore_axis_name="s")
    run = pl.kernel(
        _mf_body,
        out_type=jax.ShapeDtypeStruct((BATCH,), jnp.float32),
        mesh=mesh,
        scratch_types=[
            pltpu.VMEM((BPW,), jnp.int32),
            pltpu.VMEM((BPW,), jnp.int32),
            pltpu.VMEM((BPW,), jnp.int32),
            pltpu.VMEM((BPW,), jnp.int32),
            pltpu.VMEM((HALF, PACK * N_FACTORS), jnp.float32),
            pltpu.VMEM((HALF, PACK * N_FACTORS), jnp.float32),
            pltpu.VMEM((BPW,), jnp.float32),
            pltpu.SemaphoreType.DMA,
        ],
        compiler_params=pltpu.CompilerParams(
            needs_layout_passes=False, use_tc_tiling_on_sc=False),
    )
    return run(uf_lin, if_lin, users, items)


# speedup vs baseline: 3.1786x; 1.2330x over previous
import jax
import jax.numpy as jnp
from jax import lax
from jax.experimental import pallas as pl
from jax.experimental.pallas import tpu as pltpu
from jax.experimental.pallas import tpu_sc as plsc

NUM_ROWS = 1000000
N_FACTORS = 32
BATCH = 16384
PACK = 128 // N_FACTORS
TBLK = 16384
NBLK = (NUM_ROWS + TBLK - 1) // TBLK
SUB = TBLK // PACK
NLINES = NBLK * SUB

_info = plsc.get_sparse_core_info()
NC, NS, L = _info.num_cores, _info.num_subcores, _info.num_lanes
NW = NC * NS
BPW = BATCH // NW
CHUNK = 128
HALF = 256


def _pack_body(u_ref, v_ref, wo_ref):
    row = lax.broadcasted_iota(jnp.int32, (N_FACTORS, N_FACTORS), 0)
    col = lax.broadcasted_iota(jnp.int32, (N_FACTORS, N_FACTORS), 1)
    ident = jnp.where(row == col, 1.0, 0.0).astype(jnp.float32)
    dn = (((0,), (0,)), ((), ()))
    u = u_ref[...]
    v = v_ref[...]
    for a in range(PACK):
        us = lax.slice(u, (0, a * SUB), (N_FACTORS, (a + 1) * SUB))
        vs = lax.slice(v, (0, a * SUB), (N_FACTORS, (a + 1) * SUB))
        ut = lax.dot_general(us, ident, dn, preferred_element_type=jnp.float32)
        vt = lax.dot_general(vs, ident, dn, preferred_element_type=jnp.float32)
        ui = lax.convert_element_type(
            lax.bitcast_convert_type(
                lax.convert_element_type(ut, jnp.bfloat16), jnp.uint16),
            jnp.uint32)
        vi = lax.convert_element_type(
            lax.bitcast_convert_type(
                lax.convert_element_type(vt, jnp.bfloat16), jnp.uint16),
            jnp.uint32)
        w = ui | (vi << 16)
        wo_ref[:, a * N_FACTORS:(a + 1) * N_FACTORS] = (
            lax.bitcast_convert_type(w, jnp.float32))


def _relayout_pack(uf_t, if_t):
    ispec = pl.BlockSpec((N_FACTORS, TBLK), lambda i: (0, i))
    ospec = pl.BlockSpec((SUB, PACK * N_FACTORS), lambda i: (i, 0))
    oshape = jax.ShapeDtypeStruct((NLINES, PACK * N_FACTORS), jnp.float32)
    return pl.pallas_call(
        _pack_body,
        grid=(NBLK,),
        in_specs=[ispec, ispec],
        out_specs=ospec,
        out_shape=oshape,
    )(uf_t, if_t)


def _mf_body(w_hbm, users_hbm, items_hbm, out_hbm,
             uidx_v, iidx_v, ulns_v, ilns_v, urow_v, irow_v, out_v, sem):
    wid = lax.axis_index("s") * NC + lax.axis_index("c")
    base = wid * BPW

    pltpu.sync_copy(users_hbm.at[pl.ds(base, BPW)], uidx_v)
    pltpu.sync_copy(items_hbm.at[pl.ds(base, BPW)], iidx_v)

    def line_body(j, _):
        sl = pl.ds(j * L, L)
        u = uidx_v[sl]
        v = iidx_v[sl]
        ulns_v[sl] = (u >> 14) * SUB + (u & (SUB - 1))
        ilns_v[sl] = (v >> 14) * SUB + (v & (SUB - 1))
        return 0
    lax.fori_loop(0, BPW // L, line_body, 0)

    lane = lax.iota(jnp.int32, L)
    himask = jnp.full((L,), -65536, jnp.int32)

    def half_body(h, _):
        hbase = h * HALF
        copies = []
        for k in range(HALF // CHUNK):
            src = pl.ds(hbase + k * CHUNK, CHUNK)
            dst = pl.ds(k * CHUNK, CHUNK)
            copies.append(pltpu.async_copy(w_hbm.at[ulns_v.at[src]],
                                           urow_v.at[dst], sem))
            copies.append(pltpu.async_copy(w_hbm.at[ilns_v.at[src]],
                                           irow_v.at[dst], sem))
        for c in copies:
            c.wait()

        def group_body(g, _):
            rows = g * L + lane
            ucol = ((uidx_v[pl.ds(hbase + g * L, L)] >> 12) & 3) * N_FACTORS
            icol = ((iidx_v[pl.ds(hbase + g * L, L)] >> 12) & 3) * N_FACTORS
            acc = jnp.zeros((L,), jnp.float32)
            for d in range(N_FACTORS):
                wu = plsc.bitcast(
                    plsc.load_gather(urow_v, [rows, ucol + d]), jnp.int32)
                wi = plsc.bitcast(
                    plsc.load_gather(irow_v, [rows, icol + d]), jnp.int32)
                uu = plsc.bitcast(wu << 16, jnp.float32)
                vv = plsc.bitcast(wi & himask, jnp.float32)
                acc = acc + uu * vv
            out_v[pl.ds(hbase + g * L, L)] = acc
            return 0

        lax.fori_loop(0, HALF // L, group_body, 0)
        return 0

    lax.fori_loop(0, BPW // HALF, half_body, 0)

    pltpu.sync_copy(out_v, out_hbm.at[pl.ds(base, BPW)])


@jax.jit
def kernel(user_factors, item_factors, users, items):
    users = users.astype(jnp.int32)
    items = items.astype(jnp.int32)
    w_packed = _relayout_pack(user_factors.T, item_factors.T)
    mesh = plsc.VectorSubcoreMesh(core_axis_name="c", subcore_axis_name="s")
    run = pl.kernel(
        _mf_body,
        out_type=jax.ShapeDtypeStruct((BATCH,), jnp.float32),
        mesh=mesh,
        scratch_types=[
            pltpu.VMEM((BPW,), jnp.int32),
            pltpu.VMEM((BPW,), jnp.int32),
            pltpu.VMEM((BPW,), jnp.int32),
            pltpu.VMEM((BPW,), jnp.int32),
            pltpu.VMEM((HALF, PACK * N_FACTORS), jnp.float32),
            pltpu.VMEM((HALF, PACK * N_FACTORS), jnp.float32),
            pltpu.VMEM((BPW,), jnp.float32),
            pltpu.SemaphoreType.DMA,
        ],
        compiler_params=pltpu.CompilerParams(
            needs_layout_passes=False, use_tc_tiling_on_sc=False),
    )
    return run(w_packed, users, items)
